# baseline (speedup 1.0000x reference)
import jax
import jax.numpy as jnp
from jax import lax
from jax.experimental import pallas as pl
from jax.experimental.pallas import tpu as pltpu
from jax.experimental.pallas import tpu_sc as plsc

RES = 50
NC, NS, L = 2, 16, 16
NW = NC * NS
CHUNK = 2048
BLK = 128
CB_ROWS = 2500


def _sc_body(n_per_w, pts_hbm, cb_hbm, out_hbm, cb_v, in_v, out_v,
             sem_cb, sem_in, sem_out):
    wid = lax.axis_index("s") * NC + lax.axis_index("c")
    base = wid * n_per_w
    n_chunks = n_per_w // CHUNK

    def in_copy(c, slot):
        off = base + c * CHUNK
        return pltpu.make_async_copy(
            pts_hbm.at[pl.ds(2 * off, 2 * CHUNK)],
            in_v.at[pl.ds(slot * 2 * CHUNK, 2 * CHUNK)], sem_in)

    def out_copy(c, slot):
        off = base + c * CHUNK
        return pltpu.make_async_copy(
            out_v.at[pl.ds(slot * CHUNK, CHUNK)],
            out_hbm.at[pl.ds(off, CHUNK)], sem_out)

    cb_copy = pltpu.make_async_copy(cb_hbm, cb_v, sem_cb)
    cb_copy.start()
    in_copy(0, 0).start()
    in_copy(1, 1).start()
    cb_copy.wait()

    def round_body(c, carry):
        slot = lax.rem(c, 2)
        in_copy(c, slot).wait()

        @pl.when(c >= 2)
        def _():
            out_copy(c - 2, slot).wait()

        ib_base = slot * 2 * CHUNK
        ob_base = slot * CHUNK

        @plsc.parallel_loop(0, CHUNK // BLK, unroll=2)
        def _blk(b):
            for k in range(BLK // L):
                x = in_v[pl.ds(ib_base + b * 2 * BLK + k * L, L)]
                y = in_v[pl.ds(ib_base + b * 2 * BLK + BLK + k * L, L)]
                xi = (x * 49.0).astype(jnp.int32)
                yi = (y * 49.0).astype(jnp.int32)
                cidx = xi + yi * RES
                out_v[pl.ds(ob_base + b * BLK + k * L, L)] = (
                    plsc.load_gather(cb_v, [cidx]))

        out_copy(c, slot).start()

        @pl.when(c + 2 < n_chunks)
        def _():
            in_copy(c + 2, slot).start()

        return carry

    lax.fori_loop(0, n_chunks, round_body, 0)
    out_copy(n_chunks - 2, 0).wait()
    out_copy(n_chunks - 1, 1).wait()


def kernel(pts, codebook_0):
    n = pts.shape[0]
    n_per_w = n // NW
    pts_flat = pts.reshape(n // BLK, BLK, 2).transpose(0, 2, 1).reshape(-1)
    cb_flat = codebook_0.reshape(-1)
    mesh = plsc.VectorSubcoreMesh(core_axis_name="c", subcore_axis_name="s")
    run = pl.kernel(
        lambda *refs: _sc_body(n_per_w, *refs),
        out_type=jax.ShapeDtypeStruct((n,), jnp.float32),
        mesh=mesh,
        scratch_types=[
            pltpu.VMEM((CB_ROWS,), jnp.float32),
            pltpu.VMEM((2 * 2 * CHUNK,), jnp.float32),
            pltpu.VMEM((2 * CHUNK,), jnp.float32),
            pltpu.SemaphoreType.DMA,
            pltpu.SemaphoreType.DMA,
            pltpu.SemaphoreType.DMA,
        ],
        compiler_params=pltpu.CompilerParams(needs_layout_passes=False),
    )
    out = run(pts_flat, cb_flat)
    return out.reshape(n, 1)


# speedup vs baseline: 1.1296x; 1.1296x over previous
import jax
import jax.numpy as jnp
from jax import lax
from jax.experimental import pallas as pl
from jax.experimental.pallas import tpu as pltpu
from jax.experimental.pallas import tpu_sc as plsc

RES = 50
NC, NS, L = 2, 16, 16
NW = NC * NS
CHUNK = 8192
BLK = 128
CB_ROWS = 2500


def _sc_body(n_per_w, pts_hbm, cb_hbm, out_hbm, cb_v, in_v, out_v,
             sem_cb, sem_in, sem_out):
    wid = lax.axis_index("s") * NC + lax.axis_index("c")
    base = wid * n_per_w
    n_chunks = n_per_w // CHUNK

    def in_copy(c, slot):
        off = base + c * CHUNK
        return pltpu.make_async_copy(
            pts_hbm.at[pl.ds(2 * off, 2 * CHUNK)],
            in_v.at[pl.ds(slot * 2 * CHUNK, 2 * CHUNK)], sem_in)

    def out_copy(c, slot):
        off = base + c * CHUNK
        return pltpu.make_async_copy(
            out_v.at[pl.ds(slot * CHUNK, CHUNK)],
            out_hbm.at[pl.ds(off, CHUNK)], sem_out)

    cb_copy = pltpu.make_async_copy(cb_hbm, cb_v, sem_cb)
    cb_copy.start()
    in_copy(0, 0).start()
    in_copy(1, 1).start()
    cb_copy.wait()

    def round_body(c, carry):
        slot = lax.rem(c, 2)
        in_copy(c, slot).wait()

        @pl.when(c >= 2)
        def _():
            out_copy(c - 2, slot).wait()

        ib_base = slot * 2 * CHUNK
        ob_base = slot * CHUNK

        @plsc.parallel_loop(0, CHUNK // BLK, unroll=2)
        def _blk(b):
            for k in range(BLK // L):
                x = in_v[pl.ds(ib_base + b * 2 * BLK + k * L, L)]
                y = in_v[pl.ds(ib_base + b * 2 * BLK + BLK + k * L, L)]
                xi = (x * 49.0).astype(jnp.int32)
                yi = (y * 49.0).astype(jnp.int32)
                cidx = xi + yi * RES
                out_v[pl.ds(ob_base + b * BLK + k * L, L)] = (
                    plsc.load_gather(cb_v, [cidx]))

        out_copy(c, slot).start()

        @pl.when(c + 2 < n_chunks)
        def _():
            in_copy(c + 2, slot).start()

        return carry

    lax.fori_loop(0, n_chunks, round_body, 0)
    out_copy(n_chunks - 2, 0).wait()
    out_copy(n_chunks - 1, 1).wait()


def kernel(pts, codebook_0):
    n = pts.shape[0]
    n_per_w = n // NW
    pts_flat = pts.reshape(n // BLK, BLK, 2).transpose(0, 2, 1).reshape(-1)
    cb_flat = codebook_0.reshape(-1)
    mesh = plsc.VectorSubcoreMesh(core_axis_name="c", subcore_axis_name="s")
    run = pl.kernel(
        lambda *refs: _sc_body(n_per_w, *refs),
        out_type=jax.ShapeDtypeStruct((n,), jnp.float32),
        mesh=mesh,
        scratch_types=[
            pltpu.VMEM((CB_ROWS,), jnp.float32),
            pltpu.VMEM((2 * 2 * CHUNK,), jnp.float32),
            pltpu.VMEM((2 * CHUNK,), jnp.float32),
            pltpu.SemaphoreType.DMA,
            pltpu.SemaphoreType.DMA,
            pltpu.SemaphoreType.DMA,
        ],
        compiler_params=pltpu.CompilerParams(needs_layout_passes=False),
    )
    out = run(pts_flat, cb_flat)
    return out.reshape(n, 1)
